# baseline (speedup 1.0000x reference)
import functools

import jax
import jax.numpy as jnp
from jax import lax
from jax.experimental import pallas as pl
from jax.experimental.pallas import tpu as pltpu
from jax.experimental.pallas import tpu_sc as plsc

NUM_ROWS = 1000000
DIM = 64
PDIM = 128
BATCH = 4096
SEQ = 200
TOTAL = BATCH * SEQ

NC, NS = 2, 16
NW = NC * NS

NCOL = NUM_ROWS // PDIM
TAIL = NUM_ROWS - NCOL * PDIM
PROWS = (NCOL + 1) * PDIM
COLS_PER_W = (NCOL + NW - 1) // NW

PER_W = TOTAL // NW
CHUNK = 128
NCHUNK = PER_W // CHUNK
NBUF = 5
K = 3


def _sc_relayout(wt, tail_p):
    mesh = plsc.VectorSubcoreMesh(core_axis_name="c", subcore_axis_name="s")

    @functools.partial(
        pl.kernel,
        out_type=jax.ShapeDtypeStruct((PROWS, PDIM), jnp.float32),
        mesh=mesh,
        scratch_types=[
            pltpu.VMEM((DIM, PDIM), jnp.float32),
            pltpu.VMEM((PDIM, PDIM), jnp.float32),
        ],
        compiler_params=pltpu.CompilerParams(needs_layout_passes=False),
    )
    def ka(wt_hbm, tail_hbm, wp_hbm, cbuf, tbuf):
        wid = lax.axis_index("s") * NC + lax.axis_index("c")

        lanes = lax.iota(jnp.int32, 16)

        def body(i, carry):
            c = wid + i * NW

            @pl.when(c < NCOL)
            def _():
                pltpu.sync_copy(wt_hbm.at[:, pl.ds(c * PDIM, PDIM)], cbuf)
                for d in range(DIM):
                    cols = jnp.full((16,), d, jnp.int32)
                    for g in range(PDIM // 16):
                        v = cbuf[d, pl.ds(g * 16, 16)]
                        plsc.store_scatter(tbuf, [lanes + g * 16, cols], v)
                pltpu.sync_copy(tbuf, wp_hbm.at[pl.ds(c * PDIM, PDIM)])

            return carry

        lax.fori_loop(0, COLS_PER_W, body, 0)

        @pl.when(wid == 0)
        def _():
            pltpu.sync_copy(tail_hbm, tbuf.at[pl.ds(0, TAIL)])
            pltpu.sync_copy(
                tbuf.at[pl.ds(0, TAIL)], wp_hbm.at[pl.ds(NCOL * PDIM, TAIL)]
            )

    return ka(wt, tail_p)


def _sc_gather(idx_flat, w_pad):
    mesh = plsc.VectorSubcoreMesh(core_axis_name="c", subcore_axis_name="s")

    @functools.partial(
        pl.kernel,
        out_type=jax.ShapeDtypeStruct((TOTAL, PDIM), jnp.float32),
        mesh=mesh,
        scratch_types=[
            pltpu.VMEM((NCHUNK, CHUNK), jnp.int32),
            pltpu.VMEM((NBUF, CHUNK, PDIM), jnp.float32),
            [pltpu.SemaphoreType.DMA] * NBUF,
            [pltpu.SemaphoreType.DMA] * NBUF,
        ],
        compiler_params=pltpu.CompilerParams(use_tc_tiling_on_sc=False),
    )
    def kb(idx_hbm, w_hbm, out_hbm, idx_v, buf, gsem, wsem):
        wid = lax.axis_index("s") * NC + lax.axis_index("c")
        base = wid * PER_W
        pltpu.sync_copy(idx_hbm.at[wid], idx_v)

        def gather_start(j, b):
            pltpu.async_copy(w_hbm.at[idx_v.at[j]], buf.at[b], gsem[b])

        def gather_wait(b):
            pltpu.make_async_copy(
                w_hbm.at[pl.ds(0, CHUNK)], buf.at[b], gsem[b]
            ).wait()

        def write_start(j, b):
            pltpu.async_copy(
                buf.at[b], out_hbm.at[pl.ds(base + j * CHUNK, CHUNK)], wsem[b]
            )

        def write_wait(b):
            pltpu.make_async_copy(
                buf.at[b], out_hbm.at[pl.ds(base, CHUNK)], wsem[b]
            ).wait()

        for b in range(K):
            gather_start(b, b)
        for b in range(K, NBUF):
            gather_start(b, b)
            gather_wait(b - K)
            write_start(b - K, b - K)

        def outer(g, carry):
            for bb in range(NBUF):
                j = g * NBUF + bb
                bw = (bb + NBUF - K) % NBUF
                write_wait(bb)
                gather_start(j, bb)
                gather_wait(bw)
                write_start(j - K, bw)
            return carry

        lax.fori_loop(1, NCHUNK // NBUF, outer, 0)

        for jw in range(NCHUNK - K, NCHUNK):
            b = jw % NBUF
            gather_wait(b)
            write_start(jw, b)
        for b in range(NBUF):
            write_wait(b)

    return kb(idx_flat, w_pad)


def kernel(token_ids, W):
    tail_p = jnp.pad(W[NCOL * PDIM :, :], ((0, 0), (0, PDIM - DIM)))
    wp = _sc_relayout(W.T, tail_p)
    idx = token_ids.astype(jnp.int32).reshape(NW, NCHUNK, CHUNK)
    out = _sc_gather(idx, wp)
    return out[:, :DIM].reshape(BATCH, SEQ, DIM)


# speedup vs baseline: 2.1790x; 2.1790x over previous
import functools

import jax
import jax.numpy as jnp
from jax import lax
from jax.experimental import pallas as pl
from jax.experimental.pallas import tpu as pltpu
from jax.experimental.pallas import tpu_sc as plsc

NUM_ROWS = 1000000
DIM = 64
PDIM = 128
BATCH = 4096
SEQ = 200
TOTAL = BATCH * SEQ

NC, NS = 2, 16
NW = NC * NS

NCOL = NUM_ROWS // PDIM
TAIL = NUM_ROWS - NCOL * PDIM
PROWS = (NCOL + 1) * PDIM
COLS_PER_W = (NCOL + NW - 1) // NW

PER_W = TOTAL // NW
CHUNK = 128
NCHUNK = PER_W // CHUNK
NBUF = 5
K = 3


NFULL = 244
NEXTRA = NCOL - NFULL * NW


def _sc_relayout(wt, tail_p):
    mesh = plsc.VectorSubcoreMesh(core_axis_name="c", subcore_axis_name="s")

    @functools.partial(
        pl.kernel,
        out_type=jax.ShapeDtypeStruct((PROWS, PDIM), jnp.float32),
        mesh=mesh,
        scratch_types=[
            pltpu.VMEM((DIM, PDIM), jnp.float32),
            pltpu.VMEM((DIM, PDIM), jnp.float32),
            pltpu.VMEM((PDIM, PDIM), jnp.float32),
            pltpu.VMEM((PDIM, PDIM), jnp.float32),
            [pltpu.SemaphoreType.DMA] * 2,
            [pltpu.SemaphoreType.DMA] * 2,
        ],
        compiler_params=pltpu.CompilerParams(needs_layout_passes=False),
    )
    def ka(wt_hbm, tail_hbm, wp_hbm, cb0, cb1, tb0, tb1, rsem, wsem):
        wid = lax.axis_index("s") * NC + lax.axis_index("c")
        cbufs = [cb0, cb1]
        tbufs = [tb0, tb1]

        lanes = lax.iota(jnp.int32, 16)

        def read_start(i, par):
            c = wid + i * NW
            pltpu.async_copy(
                wt_hbm.at[:, pl.ds(c * PDIM, PDIM)], cbufs[par], rsem[par]
            )

        def read_wait(par):
            pltpu.make_async_copy(
                wt_hbm.at[:, pl.ds(0, PDIM)], cbufs[par], rsem[par]
            ).wait()

        def write_start(i, par):
            c = wid + i * NW
            pltpu.async_copy(
                tbufs[par], wp_hbm.at[pl.ds(c * PDIM, PDIM)], wsem[par]
            )

        def write_wait(par):
            pltpu.make_async_copy(
                tbufs[par], wp_hbm.at[pl.ds(0, PDIM)], wsem[par]
            ).wait()

        def transpose(cb, tb):
            def tp(p, carry):
                rot = (lanes + p) & 15
                for d0 in range(0, DIM, 16):
                    dsel = d0 + rot
                    for g in range(PDIM // 16):
                        tsel = g * 16 + lanes
                        v = plsc.load_gather(cb, [dsel, tsel])
                        plsc.store_scatter(tb, [tsel, dsel], v)
                return carry

            lax.fori_loop(0, 16, tp, 0)

        read_start(0, 0)
        read_start(1, 1)

        def outer(g, carry):
            for par in range(2):
                i = g * 2 + par
                read_wait(par)

                @pl.when(i >= 2)
                def _():
                    write_wait(par)

                transpose(cbufs[par], tbufs[par])
                write_start(i, par)

                @pl.when(i + 2 < NFULL)
                def _():
                    read_start(i + 2, par)

            return carry

        lax.fori_loop(0, NFULL // 2, outer, 0)
        write_wait(0)
        write_wait(1)

        @pl.when(wid < NEXTRA)
        def _():
            c = NFULL * NW + wid
            pltpu.sync_copy(wt_hbm.at[:, pl.ds(c * PDIM, PDIM)], cb0)
            transpose(cb0, tb0)
            pltpu.sync_copy(tb0, wp_hbm.at[pl.ds(c * PDIM, PDIM)])

        @pl.when(wid == 0)
        def _():
            pltpu.sync_copy(tail_hbm, tb1.at[pl.ds(0, TAIL)])
            pltpu.sync_copy(
                tb1.at[pl.ds(0, TAIL)], wp_hbm.at[pl.ds(NCOL * PDIM, TAIL)]
            )

    return ka(wt, tail_p)


def _sc_gather(idx_flat, w_pad):
    mesh = plsc.VectorSubcoreMesh(core_axis_name="c", subcore_axis_name="s")

    @functools.partial(
        pl.kernel,
        out_type=jax.ShapeDtypeStruct((TOTAL, PDIM), jnp.float32),
        mesh=mesh,
        scratch_types=[
            pltpu.VMEM((NCHUNK, CHUNK), jnp.int32),
            pltpu.VMEM((NBUF, CHUNK, PDIM), jnp.float32),
            [pltpu.SemaphoreType.DMA] * NBUF,
            [pltpu.SemaphoreType.DMA] * NBUF,
        ],
        compiler_params=pltpu.CompilerParams(use_tc_tiling_on_sc=False),
    )
    def kb(idx_hbm, w_hbm, out_hbm, idx_v, buf, gsem, wsem):
        wid = lax.axis_index("s") * NC + lax.axis_index("c")
        base = wid * PER_W
        pltpu.sync_copy(idx_hbm.at[wid], idx_v)

        def gather_start(j, b):
            pltpu.async_copy(w_hbm.at[idx_v.at[j]], buf.at[b], gsem[b])

        def gather_wait(b):
            pltpu.make_async_copy(
                w_hbm.at[pl.ds(0, CHUNK)], buf.at[b], gsem[b]
            ).wait()

        def write_start(j, b):
            pltpu.async_copy(
                buf.at[b], out_hbm.at[pl.ds(base + j * CHUNK, CHUNK)], wsem[b]
            )

        def write_wait(b):
            pltpu.make_async_copy(
                buf.at[b], out_hbm.at[pl.ds(base, CHUNK)], wsem[b]
            ).wait()

        for b in range(K):
            gather_start(b, b)
        for b in range(K, NBUF):
            gather_start(b, b)
            gather_wait(b - K)
            write_start(b - K, b - K)

        def outer(g, carry):
            for bb in range(NBUF):
                j = g * NBUF + bb
                bw = (bb + NBUF - K) % NBUF
                write_wait(bb)
                gather_start(j, bb)
                gather_wait(bw)
                write_start(j - K, bw)
            return carry

        lax.fori_loop(1, NCHUNK // NBUF, outer, 0)

        for jw in range(NCHUNK - K, NCHUNK):
            b = jw % NBUF
            gather_wait(b)
            write_start(jw, b)
        for b in range(NBUF):
            write_wait(b)

    return kb(idx_flat, w_pad)


def kernel(token_ids, W):
    tail_p = jnp.pad(W[NCOL * PDIM :, :], ((0, 0), (0, PDIM - DIM)))
    wp = _sc_relayout(W.T, tail_p)
    idx = token_ids.astype(jnp.int32).reshape(NW, NCHUNK, CHUNK)
    out = _sc_gather(idx, wp)
    return out[:, :DIM].reshape(BATCH, SEQ, DIM)
